# Initial kernel scaffold; baseline (speedup 1.0000x reference)
import jax
import jax.numpy as jnp
from jax.experimental import pallas as pl


def kernel(rel_bias, H, W):
    raise NotImplementedError("write your pallas kernel here")


# speedup vs baseline: 24.2441x; 24.2441x over previous
import jax
import jax.numpy as jnp
from jax.experimental import pallas as pl
from jax.experimental.pallas import tpu as pltpu

_NH = 16
_S = 32
_D = 2 * _S - 1


def _bias_body(tbl_ref, out_ref, a_scratch):
    ci = pl.program_id(1)

    @pl.when(ci == 0)
    def _stage1():
        t = jax.lax.broadcasted_iota(jnp.int32, (_S * _S, _D), 0)
        a = jax.lax.broadcasted_iota(jnp.int32, (_S * _S, _D), 1)
        prow = (a == (t // _S - t % _S + (_S - 1))).astype(jnp.float32)
        a_scratch[...] = jnp.dot(prow, tbl_ref[0],
                                 preferred_element_type=jnp.float32)

    b = jax.lax.broadcasted_iota(jnp.int32, (_D, _S), 0)
    cj = jax.lax.broadcasted_iota(jnp.int32, (_D, _S), 1)
    pc = (b == (ci - cj + (_S - 1))).astype(jnp.float32)
    res = jnp.dot(a_scratch[...], pc, preferred_element_type=jnp.float32)
    out_ref[...] = res.reshape(1, _S, 1, _S, _S)


def kernel(rel_bias, H, W):
    del H, W
    tbl = jnp.transpose(rel_bias, (2, 0, 1))
    out5 = pl.pallas_call(
        _bias_body,
        grid=(_NH, _S),
        in_specs=[pl.BlockSpec((1, _D, _D), lambda h, ci: (h, 0, 0))],
        out_specs=pl.BlockSpec((1, _S, 1, _S, _S),
                               lambda h, ci: (h, 0, ci, 0, 0)),
        out_shape=jax.ShapeDtypeStruct((_NH, _S, _S, _S, _S), jnp.float32),
        scratch_shapes=[pltpu.VMEM((_S * _S, _D), jnp.float32)],
    )(tbl)
    return out5.reshape(_NH, _S * _S, _S * _S)


# speedup vs baseline: 26.4277x; 1.0901x over previous
import jax
import jax.numpy as jnp
from jax.experimental import pallas as pl
from jax.experimental.pallas import tpu as pltpu

_NH = 16
_S = 32
_D = 2 * _S - 1


def _bias_body(tbl_ref, out_ref, q_scratch):
    ri = pl.program_id(1)

    @pl.when(ri == 0)
    def _stage1():
        tbl = tbl_ref[0]
        b = jax.lax.broadcasted_iota(jnp.int32, (_D, _S), 0)
        cj = jax.lax.broadcasted_iota(jnp.int32, (_D, _S), 1)
        for ci in range(_S):
            pc = (b == (ci - cj + (_S - 1))).astype(jnp.float32)
            q_scratch[ci] = jnp.dot(tbl, pc, preferred_element_type=jnp.float32)

    out_ref[...] = q_scratch[:, pl.ds(_S - 1 - ri, _S), :].reshape(
        1, 1, _S, _S, _S)


def kernel(rel_bias, H, W):
    del H, W
    tbl = jnp.transpose(rel_bias, (2, 0, 1))[:, ::-1, :]
    out5 = pl.pallas_call(
        _bias_body,
        grid=(_NH, _S),
        in_specs=[pl.BlockSpec((1, _D, _D), lambda h, ri: (h, 0, 0))],
        out_specs=pl.BlockSpec((1, 1, _S, _S, _S),
                               lambda h, ri: (h, ri, 0, 0, 0)),
        out_shape=jax.ShapeDtypeStruct((_NH, _S, _S, _S, _S), jnp.float32),
        scratch_shapes=[pltpu.VMEM((_S, _D, _S), jnp.float32)],
    )(tbl)
    return out5.reshape(_NH, _S * _S, _S * _S)


# speedup vs baseline: 36.6532x; 1.3869x over previous
import jax
import jax.numpy as jnp
from jax.experimental import pallas as pl
from jax.experimental.pallas import tpu as pltpu

_NH = 16
_S = 32
_D = 2 * _S - 1


_RB = 4


def _bias_body(tbl_ref, out_ref, q_scratch):
    rg = pl.program_id(1)

    @pl.when(rg == 0)
    def _stage1():
        tbl = tbl_ref[0]
        b = jax.lax.broadcasted_iota(jnp.int32, (_D, _S), 0)
        cj = jax.lax.broadcasted_iota(jnp.int32, (_D, _S), 1)
        for ci in range(_S):
            pc = (b == (ci - cj + (_S - 1))).astype(jnp.float32)
            q_scratch[ci] = jnp.dot(tbl, pc, preferred_element_type=jnp.float32)

    for r in range(_RB):
        ri = rg * _RB + r
        out_ref[0, r] = q_scratch[:, pl.ds(_S - 1 - ri, _S), :]


def kernel(rel_bias, H, W):
    del H, W
    tbl = jnp.transpose(rel_bias, (2, 0, 1))[:, ::-1, :]
    out5 = pl.pallas_call(
        _bias_body,
        grid=(_NH, _S // _RB),
        in_specs=[pl.BlockSpec((1, _D, _D), lambda h, rg: (h, 0, 0))],
        out_specs=pl.BlockSpec((1, _RB, _S, _S, _S),
                               lambda h, rg: (h, rg, 0, 0, 0)),
        out_shape=jax.ShapeDtypeStruct((_NH, _S, _S, _S, _S), jnp.float32),
        scratch_shapes=[pltpu.VMEM((_S, _D, _S), jnp.float32)],
    )(tbl)
    return out5.reshape(_NH, _S * _S, _S * _S)


# speedup vs baseline: 37.7431x; 1.0297x over previous
import jax
import jax.numpy as jnp
from jax.experimental import pallas as pl
from jax.experimental.pallas import tpu as pltpu

_NH = 16
_S = 32
_D = 2 * _S - 1


def _bias_body(tbl_ref, out_ref, q_scratch, sems):
    h = pl.program_id(0)

    tbl = tbl_ref[0]
    b = jax.lax.broadcasted_iota(jnp.int32, (_D, _S), 0)
    cj = jax.lax.broadcasted_iota(jnp.int32, (_D, _S), 1)
    for ci in range(_S):
        pc = (b == (ci - cj + (_S - 1))).astype(jnp.float32)
        q_scratch[ci] = jnp.dot(tbl, pc, preferred_element_type=jnp.float32)

    copies = []
    for ri in range(_S):
        cp = pltpu.make_async_copy(
            q_scratch.at[:, pl.ds(_S - 1 - ri, _S), :],
            out_ref.at[h, ri],
            sems.at[ri],
        )
        cp.start()
        copies.append(cp)
    for cp in copies:
        cp.wait()


def kernel(rel_bias, H, W):
    del H, W
    tbl = jnp.transpose(rel_bias, (2, 0, 1))[:, ::-1, :]
    out5 = pl.pallas_call(
        _bias_body,
        grid=(_NH,),
        in_specs=[pl.BlockSpec((1, _D, _D), lambda h: (h, 0, 0))],
        out_specs=pl.BlockSpec(memory_space=pl.MemorySpace.ANY),
        out_shape=jax.ShapeDtypeStruct((_NH, _S, _S, _S, _S), jnp.float32),
        scratch_shapes=[
            pltpu.VMEM((_S, _D, _S), jnp.float32),
            pltpu.SemaphoreType.DMA((_S,)),
        ],
    )(tbl)
    return out5.reshape(_NH, _S * _S, _S * _S)
